# baseline (speedup 1.0000x reference)
import functools

import jax
import jax.numpy as jnp
from jax import lax
from jax.experimental import pallas as pl
from jax.experimental.pallas import tpu as pltpu
from jax.experimental.pallas import tpu_sc as plsc

B_ = 64
NIMG = 192
C_ = 96
HW = 256
NCOMP = 512
D = C_ * HW

NB = 16

_GW = 24
_NCHUNK = 48
_CR = 4


def _sc_gather(table, idx_pad):
    mesh = plsc.VectorSubcoreMesh(core_axis_name="c", subcore_axis_name="s")

    @functools.partial(
        pl.kernel,
        mesh=mesh,
        out_type=jax.ShapeDtypeStruct((_NCHUNK, _CR, D), jnp.float32),
        scratch_types=[
            pltpu.VMEM((_CR,), jnp.int32),
            pltpu.VMEM((_CR, D), jnp.float32),
            pltpu.SemaphoreType.DMA,
        ],
    )
    def k(table_hbm, idx_hbm, out_hbm, idx_v, rows_v, sem):
        wid = lax.axis_index("s") * 2 + lax.axis_index("c")

        @pl.when(wid < _GW)
        def _():
            for h in range(_NCHUNK // _GW):
                j = wid * 2 + h
                pltpu.sync_copy(idx_hbm.at[pl.ds(8 * j, _CR)], idx_v)
                pltpu.async_copy(table_hbm.at[idx_v], rows_v, sem).wait()
                pltpu.sync_copy(rows_v, out_hbm.at[j])

    return k(table, idx_pad)


CP = 128


def _shift_rows(a, s):
    if s == 0:
        return a
    n, _, c = a.shape
    if s > 0:
        pad = jnp.zeros((n, s, c), a.dtype)
        return jnp.concatenate([a[:, s:, :], pad], axis=1)
    pad = jnp.zeros((n, -s, c), a.dtype)
    return jnp.concatenate([pad, a[:, :HW + s, :]], axis=1)


def _conv_body(g_ref, x_ref, w_ref, b_ref, o_ref):
    nb = g_ref.shape[0]
    a = jnp.transpose(g_ref[...], (0, 2, 1))
    a = jnp.concatenate(
        [a, jnp.zeros((nb, HW, CP - C_), jnp.float32)], axis=2)
    p = lax.broadcasted_iota(jnp.int32, (1, HW, 1), 1)
    hh = p // 16
    ww = p % 16
    for l in range(3):
        ab = jnp.maximum(a, 0.0).astype(jnp.bfloat16)
        pieces = []
        for ky in range(3):
            for kx in range(3):
                s = (ky - 1) * 16 + (kx - 1)
                m = ((hh + (ky - 1) >= 0) & (hh + (ky - 1) < 16)
                     & (ww + (kx - 1) >= 0) & (ww + (kx - 1) < 16))
                pieces.append(jnp.where(m, _shift_rows(ab, s),
                                        jnp.bfloat16(0.0)))
        t = jnp.concatenate(pieces, axis=2)
        mm = jnp.dot(t.reshape(nb * HW, 9 * CP), w_ref[l],
                     preferred_element_type=jnp.float32)
        a = mm.reshape(nb, HW, CP) + b_ref[l][None, None, :]
    acc = jnp.transpose(a, (0, 2, 1))
    o_ref[...] = acc[:, :C_, :]


def _conv_call(g, xr, wt, bs):
    grid = (NIMG // NB,)
    return pl.pallas_call(
        _conv_body,
        grid=grid,
        in_specs=[
            pl.BlockSpec((NB, C_, HW), lambda i: (i, 0, 0)),
            pl.BlockSpec((NB, C_, HW), lambda i: (i, 0, 0)),
            pl.BlockSpec((3, 9 * CP, CP), lambda i: (0, 0, 0)),
            pl.BlockSpec((3, CP), lambda i: (0, 0)),
        ],
        out_specs=pl.BlockSpec((NB, C_, HW), lambda i: (i, 0, 0)),
        out_shape=jax.ShapeDtypeStruct((NIMG, C_, HW), jnp.float32),
        compiler_params=pltpu.CompilerParams(
            dimension_semantics=("parallel",)),
    )(g, xr, wt, bs)


def kernel(x, comp_addrs, bias, W1, b1, W2, b2, W3, b3):
    addrs = comp_addrs.reshape(NIMG).astype(jnp.int32)
    addrs_p = jnp.pad(addrs.reshape(_NCHUNK, _CR),
                      ((0, 0), (0, 8 - _CR))).reshape(_NCHUNK * 8)
    g = _sc_gather(bias.reshape(NCOMP, D), addrs_p).reshape(NIMG, C_, HW)
    wt = jnp.stack([W1, W2, W3]).transpose(0, 3, 4, 2, 1)
    wt = jnp.pad(wt, ((0, 0), (0, 0), (0, 0), (0, CP - C_), (0, CP - C_)))
    wt = wt.reshape(3, 9 * CP, CP).astype(jnp.bfloat16)
    bs = jnp.pad(jnp.stack([b1, b2, b3]), ((0, 0), (0, CP - C_)))
    yt = _conv_call(g, x.reshape(NIMG, C_, HW), wt, bs)
    return yt.reshape(B_, 3, C_, 16, 16)


# speedup vs baseline: 1.1221x; 1.1221x over previous
import functools

import jax
import jax.numpy as jnp
from jax import lax
from jax.experimental import pallas as pl
from jax.experimental.pallas import tpu as pltpu
from jax.experimental.pallas import tpu_sc as plsc

B_ = 64
NIMG = 192
C_ = 96
HW = 256
NCOMP = 512
D = C_ * HW

NB = 16

_GW = 24
_NCHUNK = 48
_CR = 4


def _sc_gather(table, idx_pad):
    mesh = plsc.VectorSubcoreMesh(core_axis_name="c", subcore_axis_name="s")

    @functools.partial(
        pl.kernel,
        mesh=mesh,
        out_type=jax.ShapeDtypeStruct((_NCHUNK, _CR, D), jnp.float32),
        scratch_types=[
            pltpu.VMEM((_CR,), jnp.int32),
            pltpu.VMEM((_CR, D), jnp.float32),
            pltpu.SemaphoreType.DMA,
        ],
    )
    def k(table_hbm, idx_hbm, out_hbm, idx_v, rows_v, sem):
        wid = lax.axis_index("s") * 2 + lax.axis_index("c")

        @pl.when(wid < _GW)
        def _():
            for h in range(_NCHUNK // _GW):
                j = wid * 2 + h
                pltpu.sync_copy(idx_hbm.at[pl.ds(8 * j, _CR)], idx_v)
                pltpu.async_copy(table_hbm.at[idx_v], rows_v, sem).wait()
                pltpu.sync_copy(rows_v, out_hbm.at[j])

    return k(table, idx_pad)


CP = 128


def _shift_rows(a, s):
    if s == 0:
        return a
    n, _, c = a.shape
    if s > 0:
        pad = jnp.zeros((n, s, c), a.dtype)
        return jnp.concatenate([a[:, s:, :], pad], axis=1)
    pad = jnp.zeros((n, -s, c), a.dtype)
    return jnp.concatenate([pad, a[:, :HW + s, :]], axis=1)


def _conv_body(g_ref, x_ref, w_ref, b_ref, o_ref, tbuf):
    nb = g_ref.shape[0]
    a = jnp.transpose(g_ref[...], (0, 2, 1))
    a = jnp.concatenate(
        [a, jnp.zeros((nb, HW, CP - C_), jnp.float32)], axis=2)
    ww = lax.broadcasted_iota(jnp.int32, (1, HW, 1), 1) % 16
    tbuf[:, 0:16, :] = jnp.zeros((nb, 16, 3 * CP), jnp.bfloat16)
    tbuf[:, HW + 16:HW + 32, :] = jnp.zeros((nb, 16, 3 * CP), jnp.bfloat16)
    for l in range(3):
        ab = jnp.maximum(a, 0.0).astype(jnp.bfloat16)
        for kx in range(3):
            m = (ww + (kx - 1) >= 0) & (ww + (kx - 1) < 16)
            tbuf[:, 16:HW + 16, kx * CP:(kx + 1) * CP] = jnp.where(
                m, _shift_rows(ab, kx - 1), jnp.bfloat16(0.0))
        acc = b_ref[l][None, None, :]
        for ky in range(3):
            tb = tbuf[:, 16 * ky:16 * ky + HW, :]
            acc = acc + jnp.dot(
                tb.reshape(nb * HW, 3 * CP), w_ref[l, ky],
                preferred_element_type=jnp.float32).reshape(nb, HW, CP)
        a = acc
    acc = jnp.transpose(a, (0, 2, 1))
    o_ref[...] = x_ref[...] + acc[:, :C_, :]


def _conv_call(g, xr, wt, bs):
    grid = (NIMG // NB,)
    return pl.pallas_call(
        _conv_body,
        grid=grid,
        in_specs=[
            pl.BlockSpec((NB, C_, HW), lambda i: (i, 0, 0)),
            pl.BlockSpec((NB, C_, HW), lambda i: (i, 0, 0)),
            pl.BlockSpec((3, 3, 3 * CP, CP), lambda i: (0, 0, 0, 0)),
            pl.BlockSpec((3, CP), lambda i: (0, 0)),
        ],
        out_specs=pl.BlockSpec((NB, C_, HW), lambda i: (i, 0, 0)),
        out_shape=jax.ShapeDtypeStruct((NIMG, C_, HW), jnp.float32),
        scratch_shapes=[pltpu.VMEM((NB, HW + 32, 3 * CP), jnp.bfloat16)],
        compiler_params=pltpu.CompilerParams(
            dimension_semantics=("parallel",)),
    )(g, xr, wt, bs)


def kernel(x, comp_addrs, bias, W1, b1, W2, b2, W3, b3):
    addrs = comp_addrs.reshape(NIMG).astype(jnp.int32)
    addrs_p = jnp.pad(addrs.reshape(_NCHUNK, _CR),
                      ((0, 0), (0, 8 - _CR))).reshape(_NCHUNK * 8)
    g = _sc_gather(bias.reshape(NCOMP, D), addrs_p).reshape(NIMG, C_, HW)
    wt = jnp.stack([W1, W2, W3]).transpose(0, 3, 4, 2, 1)
    wt = jnp.pad(wt, ((0, 0), (0, 0), (0, 0), (0, CP - C_), (0, CP - C_)))
    wt = wt.reshape(3, 3, 3 * CP, CP).astype(jnp.bfloat16)
    bs = jnp.pad(jnp.stack([b1, b2, b3]), ((0, 0), (0, CP - C_)))
    yt = _conv_call(g, x.reshape(NIMG, C_, HW), wt, bs)
    return yt.reshape(B_, 3, C_, 16, 16)


# speedup vs baseline: 1.1253x; 1.0029x over previous
import functools

import jax
import jax.numpy as jnp
from jax import lax
from jax.experimental import pallas as pl
from jax.experimental.pallas import tpu as pltpu
from jax.experimental.pallas import tpu_sc as plsc

B_ = 64
NIMG = 192
C_ = 96
HW = 256
NCOMP = 512
D = C_ * HW

NB = 32

_GW = 24
_NCHUNK = 48
_CR = 4


def _sc_gather(table, idx_pad, nchunk, out_dtype=None):
    mesh = plsc.VectorSubcoreMesh(core_axis_name="c", subcore_axis_name="s")

    @functools.partial(
        pl.kernel,
        mesh=mesh,
        out_type=jax.ShapeDtypeStruct((nchunk, _CR, D), jnp.float32),
        scratch_types=[
            pltpu.VMEM((_CR,), jnp.int32),
            pltpu.VMEM((_CR, D), jnp.float32),
            pltpu.SemaphoreType.DMA,
        ],
    )
    def k(table_hbm, idx_hbm, out_hbm, idx_v, rows_v, sem):
        wid = lax.axis_index("s") * 2 + lax.axis_index("c")
        npw = nchunk // _GW

        @pl.when(wid < _GW)
        def _():
            for h in range(npw):
                j = wid * npw + h
                pltpu.sync_copy(idx_hbm.at[pl.ds(8 * j, _CR)], idx_v)
                pltpu.async_copy(table_hbm.at[idx_v], rows_v, sem).wait()
                pltpu.sync_copy(rows_v, out_hbm.at[j])

    return k(table, idx_pad)


CP = 128


def _shift_rows(a, s):
    if s == 0:
        return a
    n, _, c = a.shape
    if s > 0:
        pad = jnp.zeros((n, s, c), a.dtype)
        return jnp.concatenate([a[:, s:, :], pad], axis=1)
    pad = jnp.zeros((n, -s, c), a.dtype)
    return jnp.concatenate([pad, a[:, :HW + s, :]], axis=1)


def _conv_body(g_ref, x_ref, w_ref, b_ref, o_ref, tbuf):
    nb = g_ref.shape[0]
    a = jnp.transpose(g_ref[...], (0, 2, 1))
    a = jnp.concatenate(
        [a, jnp.zeros((nb, HW, CP - C_), jnp.float32)], axis=2)
    ww = lax.broadcasted_iota(jnp.int32, (1, HW, 1), 1) % 16
    tbuf[:, 0:16, :] = jnp.zeros((nb, 16, 3 * CP), jnp.bfloat16)
    tbuf[:, HW + 16:HW + 32, :] = jnp.zeros((nb, 16, 3 * CP), jnp.bfloat16)
    for l in range(3):
        ab = jnp.maximum(a, 0.0).astype(jnp.bfloat16)
        for kx in range(3):
            m = (ww + (kx - 1) >= 0) & (ww + (kx - 1) < 16)
            tbuf[:, 16:HW + 16, kx * CP:(kx + 1) * CP] = jnp.where(
                m, _shift_rows(ab, kx - 1), jnp.bfloat16(0.0))
        acc = b_ref[l][None, None, :]
        for ky in range(3):
            tb = tbuf[:, 16 * ky:16 * ky + HW, :]
            acc = acc + jnp.dot(
                tb.reshape(nb * HW, 3 * CP), w_ref[l, ky],
                preferred_element_type=jnp.float32).reshape(nb, HW, CP)
        a = acc
    acc = jnp.transpose(a, (0, 2, 1))
    o_ref[...] = x_ref[...] + acc[:, :C_, :]


def _conv_call(g, xr, wt, bs):
    grid = (NIMG // NB,)
    return pl.pallas_call(
        _conv_body,
        grid=grid,
        in_specs=[
            pl.BlockSpec((NB, C_, HW), lambda i: (i, 0, 0)),
            pl.BlockSpec((NB, C_, HW), lambda i: (i, 0, 0)),
            pl.BlockSpec((3, 3, 3 * CP, CP), lambda i: (0, 0, 0, 0)),
            pl.BlockSpec((3, CP), lambda i: (0, 0)),
        ],
        out_specs=pl.BlockSpec((NB, C_, HW), lambda i: (i, 0, 0)),
        out_shape=jax.ShapeDtypeStruct((NIMG, C_, HW), jnp.float32),
        scratch_shapes=[pltpu.VMEM((NB, HW + 32, 3 * CP), jnp.bfloat16)],
        compiler_params=pltpu.CompilerParams(
            dimension_semantics=("parallel",)),
    )(g, xr, wt, bs)


def kernel(x, comp_addrs, bias, W1, b1, W2, b2, W3, b3):
    addrs = comp_addrs.reshape(NIMG).astype(jnp.int32)
    addrs_p = jnp.pad(addrs.reshape(_NCHUNK, _CR),
                      ((0, 0), (0, 8 - _CR))).reshape(_NCHUNK * 8)
    g = _sc_gather(bias.reshape(NCOMP, D), addrs_p,
                   _NCHUNK).reshape(NIMG, C_, HW)
    wt = jnp.stack([W1, W2, W3]).transpose(0, 3, 4, 2, 1)
    wt = jnp.pad(wt, ((0, 0), (0, 0), (0, 0), (0, CP - C_), (0, CP - C_)))
    wt = wt.reshape(3, 3, 3 * CP, CP).astype(jnp.bfloat16)
    bs = jnp.pad(jnp.stack([b1, b2, b3]), ((0, 0), (0, CP - C_)))
    yt = _conv_call(g, x.reshape(NIMG, C_, HW), wt, bs)
    return yt.reshape(B_, 3, C_, 16, 16)


# speedup vs baseline: 1.2440x; 1.1055x over previous
import functools

import jax
import jax.numpy as jnp
from jax import lax
from jax.experimental import pallas as pl
from jax.experimental.pallas import tpu as pltpu
from jax.experimental.pallas import tpu_sc as plsc

B_ = 64
NIMG = 192
C_ = 96
HW = 256
NCOMP = 512
D = C_ * HW

NB = 32

_GW = 24
_NCHUNK = 48
_CR = 4


def _sc_gather(table, idx_pad, nchunk):
    mesh = plsc.VectorSubcoreMesh(core_axis_name="c", subcore_axis_name="s")

    @functools.partial(
        pl.kernel,
        mesh=mesh,
        out_type=jax.ShapeDtypeStruct((nchunk * _CR, C_, HW), jnp.float32),
        scratch_types=[
            pltpu.VMEM((_CR,), jnp.int32),
            pltpu.VMEM((_CR, C_, HW), jnp.float32),
            pltpu.SemaphoreType.DMA,
        ],
    )
    def k(table_hbm, idx_hbm, out_hbm, idx_v, rows_v, sem):
        wid = lax.axis_index("s") * 2 + lax.axis_index("c")
        npw = nchunk // _GW

        @pl.when(wid < _GW)
        def _():
            for h in range(npw):
                j = wid * npw + h
                pltpu.sync_copy(idx_hbm.at[pl.ds(8 * j, _CR)], idx_v)
                pltpu.async_copy(table_hbm.at[idx_v], rows_v, sem).wait()
                pltpu.sync_copy(rows_v, out_hbm.at[pl.ds(_CR * j, _CR)])

    return k(table, idx_pad)


CP = 128


def _shift_rows(a, s):
    if s == 0:
        return a
    n, _, c = a.shape
    if s > 0:
        pad = jnp.zeros((n, s, c), a.dtype)
        return jnp.concatenate([a[:, s:, :], pad], axis=1)
    pad = jnp.zeros((n, -s, c), a.dtype)
    return jnp.concatenate([pad, a[:, :HW + s, :]], axis=1)


def _conv_body(g_ref, x_ref, w_ref, b_ref, o_ref, tbuf):
    nb = g_ref.shape[0]
    a = jnp.transpose(g_ref[...], (0, 2, 1))
    a = jnp.concatenate(
        [a, jnp.zeros((nb, HW, CP - C_), jnp.float32)], axis=2)
    ww = lax.broadcasted_iota(jnp.int32, (1, HW, 1), 1) % 16
    tbuf[:, 0:16, :] = jnp.zeros((nb, 16, 3 * CP), jnp.bfloat16)
    tbuf[:, HW + 16:HW + 32, :] = jnp.zeros((nb, 16, 3 * CP), jnp.bfloat16)
    for l in range(3):
        ab = jnp.maximum(a, 0.0).astype(jnp.bfloat16)
        for kx in range(3):
            m = (ww + (kx - 1) >= 0) & (ww + (kx - 1) < 16)
            tbuf[:, 16:HW + 16, kx * CP:(kx + 1) * CP] = jnp.where(
                m, _shift_rows(ab, kx - 1), jnp.bfloat16(0.0))
        acc = b_ref[l][None, None, :]
        for ky in range(3):
            tb = tbuf[:, 16 * ky:16 * ky + HW, :]
            acc = acc + jnp.dot(
                tb.reshape(nb * HW, 3 * CP), w_ref[l, ky],
                preferred_element_type=jnp.float32).reshape(nb, HW, CP)
        a = acc
    acc = jnp.transpose(a, (0, 2, 1))
    o_ref[...] = x_ref[...] + acc[:, :C_, :]


def _conv_call(g, xr, wt, bs):
    grid = (NIMG // NB,)
    return pl.pallas_call(
        _conv_body,
        grid=grid,
        in_specs=[
            pl.BlockSpec((NB, C_, HW), lambda i: (i, 0, 0)),
            pl.BlockSpec((NB, C_, HW), lambda i: (i, 0, 0)),
            pl.BlockSpec((3, 3, 3 * CP, CP), lambda i: (0, 0, 0, 0)),
            pl.BlockSpec((3, CP), lambda i: (0, 0)),
        ],
        out_specs=pl.BlockSpec((NB, C_, HW), lambda i: (i, 0, 0)),
        out_shape=jax.ShapeDtypeStruct((NIMG, C_, HW), jnp.float32),
        scratch_shapes=[pltpu.VMEM((NB, HW + 32, 3 * CP), jnp.bfloat16)],
        compiler_params=pltpu.CompilerParams(
            dimension_semantics=("parallel",)),
    )(g, xr, wt, bs)


def kernel(x, comp_addrs, bias, W1, b1, W2, b2, W3, b3):
    addrs = comp_addrs.reshape(NIMG).astype(jnp.int32)
    addrs_p = jnp.pad(addrs.reshape(_NCHUNK, _CR),
                      ((0, 0), (0, 8 - _CR))).reshape(_NCHUNK * 8)
    g = _sc_gather(bias.reshape(NCOMP, C_, HW), addrs_p, _NCHUNK)
    wt = jnp.stack([W1, W2, W3]).transpose(0, 3, 4, 2, 1)
    wt = jnp.pad(wt, ((0, 0), (0, 0), (0, 0), (0, CP - C_), (0, CP - C_)))
    wt = wt.reshape(3, 3, 3 * CP, CP).astype(jnp.bfloat16)
    bs = jnp.pad(jnp.stack([b1, b2, b3]), ((0, 0), (0, CP - C_)))
    yt = _conv_call(g, x.reshape(NIMG, C_, HW), wt, bs)
    return yt.reshape(B_, 3, C_, 16, 16)
